# Initial kernel scaffold; baseline (speedup 1.0000x reference)
import functools

import jax
import jax.numpy as jnp
from jax import lax
from jax.experimental import pallas as pl
from jax.experimental.pallas import tpu as pltpu
from jax.experimental.pallas import tpu_sc as plsc

N = 10000
E = 320000
IN = 128
HID = 32
HEADS = 8
OUT = 128
G = 64

NC = 2
NS = 16
NW = NC * NS
EPW = E // NW
CH = 80
NCHUNK = EPW // CH
RPT = N // NS
RB = 2000
GRID = N // RB

_f32 = jnp.float32


def _tc1_body(x_ref, wg_ref, as_ref, ad_ref, h0_ref, h1_ref, ts_ref, td_ref):
    h = jnp.dot(x_ref[...], wg_ref[...])
    h0_ref[...] = h[:, :128]
    h1_ref[...] = h[:, 128:]
    ts_ref[...] = jnp.dot(h, as_ref[...])
    td_ref[...] = jnp.dot(h, ad_ref[...])


def _tc1(x, W_gat, As, Ad):
    return pl.pallas_call(
        _tc1_body,
        grid=(GRID,),
        in_specs=[
            pl.BlockSpec((RB, IN), lambda i: (i, 0)),
            pl.BlockSpec((IN, HEADS * HID), lambda i: (0, 0)),
            pl.BlockSpec((HEADS * HID, 16), lambda i: (0, 0)),
            pl.BlockSpec((HEADS * HID, 16), lambda i: (0, 0)),
        ],
        out_specs=[
            pl.BlockSpec((RB, 128), lambda i: (i, 0)),
            pl.BlockSpec((RB, 128), lambda i: (i, 0)),
            pl.BlockSpec((RB, 16), lambda i: (i, 0)),
            pl.BlockSpec((RB, 16), lambda i: (i, 0)),
        ],
        out_shape=[
            jax.ShapeDtypeStruct((N, 128), _f32),
            jax.ShapeDtypeStruct((N, 128), _f32),
            jax.ShapeDtypeStruct((N, 16), _f32),
            jax.ShapeDtypeStruct((N, 16), _f32),
        ],
    )(x, W_gat, As, Ad)


def _tc2_body(a00_ref, a10_ref, a01_ref, a11_ref, b_ref, g0_ref, g1_ref):
    b = b_ref[...]
    g0_ref[...] = jnp.maximum(a00_ref[...] + a10_ref[...] + b[:, :128], 0.0)
    g1_ref[...] = jnp.maximum(a01_ref[...] + a11_ref[...] + b[:, 128:], 0.0)


def _tc2(a00, a10, a01, a11, b2d):
    return pl.pallas_call(
        _tc2_body,
        grid=(GRID,),
        in_specs=[pl.BlockSpec((RB, 128), lambda i: (i, 0))] * 4
        + [pl.BlockSpec((1, HEADS * HID), lambda i: (0, 0))],
        out_specs=[pl.BlockSpec((RB, 128), lambda i: (i, 0))] * 2,
        out_shape=[jax.ShapeDtypeStruct((N, 128), _f32)] * 2,
    )(a00, a10, a01, a11, b2d)


def _tc3_body(g0_ref, g1_ref, n00_ref, n01_ref, n10_ref, n11_ref, bt_ref,
              w1a_ref, w1b_ref, b1_ref, w2_ref, b2_ref, wf_ref, bf_ref,
              emb_ref, sums, counts):
    i = pl.program_id(0)

    @pl.when(i == 0)
    def _():
        sums[...] = jnp.zeros_like(sums)
        counts[...] = jnp.zeros_like(counts)

    gin0 = g0_ref[...] + n00_ref[...] + n10_ref[...]
    gin1 = g1_ref[...] + n01_ref[...] + n11_ref[...]
    h1 = jnp.maximum(
        jnp.dot(gin0, w1a_ref[...]) + jnp.dot(gin1, w1b_ref[...]) + b1_ref[...], 0.0)
    hg = jnp.maximum(jnp.dot(h1, w2_ref[...]) + b2_ref[...], 0.0)
    ids = jnp.broadcast_to(bt_ref[...], (RB, G))
    iota = lax.broadcasted_iota(jnp.int32, (RB, G), 1)
    onehot = (ids == iota).astype(_f32)
    sums[...] += lax.dot_general(onehot, hg, (((0,), (0,)), ((), ())))
    counts[...] += lax.dot_general(
        onehot, jnp.ones((RB, HID), _f32), (((0,), (0,)), ((), ())))

    @pl.when(i == GRID - 1)
    def _():
        pooled = sums[...] / jnp.maximum(counts[...], 1.0)
        emb_ref[...] = jnp.dot(pooled, wf_ref[...]) + bf_ref[...]


def _tc3(g0, g1, n00, n01, n10, n11, bt, W1a, W1b, b1, W2, b2, Wf, bf):
    return pl.pallas_call(
        _tc3_body,
        grid=(GRID,),
        in_specs=[pl.BlockSpec((RB, 128), lambda i: (i, 0))] * 6
        + [
            pl.BlockSpec((RB, 1), lambda i: (i, 0)),
            pl.BlockSpec((128, HID), lambda i: (0, 0)),
            pl.BlockSpec((128, HID), lambda i: (0, 0)),
            pl.BlockSpec((1, HID), lambda i: (0, 0)),
            pl.BlockSpec((HID, HID), lambda i: (0, 0)),
            pl.BlockSpec((1, HID), lambda i: (0, 0)),
            pl.BlockSpec((HID, OUT), lambda i: (0, 0)),
            pl.BlockSpec((1, OUT), lambda i: (0, 0)),
        ],
        out_specs=pl.BlockSpec((G, OUT), lambda i: (0, 0)),
        out_shape=jax.ShapeDtypeStruct((G, OUT), _f32),
        scratch_shapes=[pltpu.VMEM((G, HID), _f32), pltpu.VMEM((G, HID), _f32)],
    )(g0, g1, n00, n01, n10, n11, bt, W1a, W1b, b1, W2, b2, Wf, bf)


_MESH = plsc.VectorSubcoreMesh(core_axis_name="c", subcore_axis_name="s")


def _sc_a(src, dst, ast, adt, z16):
    @functools.partial(
        pl.kernel, mesh=_MESH,
        out_type=[
            jax.ShapeDtypeStruct((E, 16), _f32),
            jax.ShapeDtypeStruct((N, 16), _f32),
            jax.ShapeDtypeStruct((N, 16), _f32),
        ],
        scratch_types=[
            pltpu.VMEM((CH,), jnp.int32),
            pltpu.VMEM((CH,), jnp.int32),
            pltpu.VMEM((CH, 16), _f32),
            pltpu.VMEM((CH, 16), _f32),
            pltpu.VMEM((CH, 16), _f32),
            pltpu.VMEM_SHARED((N, 16), _f32),
            pltpu.SemaphoreType.DMA,
            pltpu.SemaphoreType.DMA,
        ],
    )
    def k(src_h, dst_h, ast_h, adt_h, z16_h, ex_h, den0_h, den1_h,
          src_v, dst_v, srows, drows, ex_v, den_sp, sem1, sem2):
        c = lax.axis_index("c")
        s = lax.axis_index("s")
        base = (c * NS + s) * EPW
        rs = pl.ds(s * RPT, RPT)
        pltpu.sync_copy(z16_h, den_sp.at[rs])
        plsc.subcore_barrier()

        def chunk(i, carry):
            off = pl.multiple_of(base + i * CH, 8)
            pltpu.sync_copy(src_h.at[pl.ds(off, CH)], src_v)
            pltpu.sync_copy(dst_h.at[pl.ds(off, CH)], dst_v)
            cp1 = pltpu.async_copy(ast_h.at[src_v], srows, sem1)
            cp2 = pltpu.async_copy(adt_h.at[dst_v], drows, sem2)
            cp1.wait()
            cp2.wait()

            def row(j, carry2):
                v = srows[j] + drows[j]
                ex_v[j] = jnp.exp(jnp.maximum(v, 0.2 * v))
                return carry2

            lax.fori_loop(0, CH, row, 0)
            pltpu.sync_copy(ex_v, den_sp.at[dst_v], add=True)
            pltpu.sync_copy(ex_v, ex_h.at[pl.ds(off, CH)])
            return carry

        lax.fori_loop(0, NCHUNK, chunk, 0)
        plsc.subcore_barrier()

        @pl.when(c == 0)
        def _():
            pltpu.sync_copy(den_sp.at[rs], den0_h.at[rs])

        @pl.when(c == 1)
        def _():
            pltpu.sync_copy(den_sp.at[rs], den1_h.at[rs])

    return k(src, dst, ast, adt, z16)


def _sc_c(dst, ex, den0, den1):
    @functools.partial(
        pl.kernel, mesh=_MESH,
        out_type=jax.ShapeDtypeStruct((E, 16), _f32),
        scratch_types=[
            pltpu.VMEM((CH,), jnp.int32),
            pltpu.VMEM((CH, 16), _f32),
            pltpu.VMEM((CH, 16), _f32),
            pltpu.VMEM((CH, 16), _f32),
            pltpu.SemaphoreType.DMA,
            pltpu.SemaphoreType.DMA,
        ],
    )
    def k(dst_h, ex_h, den0_h, den1_h, al_h,
          dst_v, ex_v, d0, d1, sem1, sem2):
        c = lax.axis_index("c")
        s = lax.axis_index("s")
        base = (c * NS + s) * EPW

        def chunk(i, carry):
            off = pl.multiple_of(base + i * CH, 8)
            pltpu.sync_copy(dst_h.at[pl.ds(off, CH)], dst_v)
            pltpu.sync_copy(ex_h.at[pl.ds(off, CH)], ex_v)
            cp1 = pltpu.async_copy(den0_h.at[dst_v], d0, sem1)
            cp2 = pltpu.async_copy(den1_h.at[dst_v], d1, sem2)
            cp1.wait()
            cp2.wait()

            def row(j, carry2):
                ex_v[j] = ex_v[j] / (d0[j] + d1[j] + 1e-16)
                return carry2

            lax.fori_loop(0, CH, row, 0)
            pltpu.sync_copy(ex_v, al_h.at[pl.ds(off, CH)])
            return carry

        lax.fori_loop(0, NCHUNK, chunk, 0)

    return k(dst, ex, den0, den1)


def _sc_d(src, dst, h0, h1, alpha, z128):
    @functools.partial(
        pl.kernel, mesh=_MESH,
        out_type=[jax.ShapeDtypeStruct((N, 128), _f32)] * 4,
        scratch_types=[
            pltpu.VMEM((CH,), jnp.int32),
            pltpu.VMEM((CH,), jnp.int32),
            pltpu.VMEM((CH, 16), _f32),
            pltpu.VMEM((CH, 128), _f32),
            pltpu.VMEM_SHARED((N, 128), _f32),
            pltpu.SemaphoreType.DMA,
        ],
    )
    def k(src_h, dst_h, h0_h, h1_h, al_h, z128_h,
          a00_h, a01_h, a10_h, a11_h,
          src_v, dst_v, al_v, hrows, agg_sp, sem1):
        c = lax.axis_index("c")
        s = lax.axis_index("s")
        base = (c * NS + s) * EPW
        rs = pl.ds(s * RPT, RPT)
        outs = ((a00_h, a01_h), (a10_h, a11_h))
        for g in range(2):
            htab = h0_h if g == 0 else h1_h
            pltpu.sync_copy(z128_h, agg_sp.at[rs])
            plsc.subcore_barrier()

            def chunk(i, carry, g=g, htab=htab):
                off = pl.multiple_of(base + i * CH, 8)
                pltpu.sync_copy(src_h.at[pl.ds(off, CH)], src_v)
                pltpu.sync_copy(dst_h.at[pl.ds(off, CH)], dst_v)
                pltpu.sync_copy(al_h.at[pl.ds(off, CH)], al_v)
                pltpu.async_copy(htab.at[src_v], hrows, sem1).wait()

                def row(e, carry2):
                    for h4 in range(4):
                        a = al_v[e, 4 * g + h4]
                        for b in range(2):
                            sl = pl.ds(h4 * 32 + b * 16, 16)
                            hrows[e, sl] = hrows[e, sl] * a
                    return carry2

                lax.fori_loop(0, CH, row, 0)
                pltpu.sync_copy(hrows, agg_sp.at[dst_v], add=True)
                return carry

            lax.fori_loop(0, NCHUNK, chunk, 0)
            plsc.subcore_barrier()

            @pl.when(c == 0)
            def _(g=g):
                pltpu.sync_copy(agg_sp.at[rs], outs[0][g].at[rs])

            @pl.when(c == 1)
            def _(g=g):
                pltpu.sync_copy(agg_sp.at[rs], outs[1][g].at[rs])

    return k(src, dst, h0, h1, alpha, z128)


def _sc_e(src, dst, g0, g1, z128):
    @functools.partial(
        pl.kernel, mesh=_MESH,
        out_type=[jax.ShapeDtypeStruct((N, 128), _f32)] * 4,
        scratch_types=[
            pltpu.VMEM((CH,), jnp.int32),
            pltpu.VMEM((CH,), jnp.int32),
            pltpu.VMEM((CH, 128), _f32),
            pltpu.VMEM_SHARED((N, 128), _f32),
            pltpu.SemaphoreType.DMA,
        ],
    )
    def k(src_h, dst_h, g0_h, g1_h, z128_h,
          n00_h, n01_h, n10_h, n11_h,
          src_v, dst_v, grows, nb_sp, sem1):
        c = lax.axis_index("c")
        s = lax.axis_index("s")
        base = (c * NS + s) * EPW
        rs = pl.ds(s * RPT, RPT)
        outs = ((n00_h, n01_h), (n10_h, n11_h))
        for g in range(2):
            gtab = g0_h if g == 0 else g1_h
            pltpu.sync_copy(z128_h, nb_sp.at[rs])
            plsc.subcore_barrier()

            def chunk(i, carry, gtab=gtab):
                off = pl.multiple_of(base + i * CH, 8)
                pltpu.sync_copy(src_h.at[pl.ds(off, CH)], src_v)
                pltpu.sync_copy(dst_h.at[pl.ds(off, CH)], dst_v)
                pltpu.async_copy(gtab.at[src_v], grows, sem1).wait()
                pltpu.sync_copy(grows, nb_sp.at[dst_v], add=True)
                return carry

            lax.fori_loop(0, NCHUNK, chunk, 0)
            plsc.subcore_barrier()

            @pl.when(c == 0)
            def _(g=g):
                pltpu.sync_copy(nb_sp.at[rs], outs[0][g].at[rs])

            @pl.when(c == 1)
            def _(g=g):
                pltpu.sync_copy(nb_sp.at[rs], outs[1][g].at[rs])

    return k(src, dst, g0, g1, z128)


def kernel(x, edge_index, batch, W_gat, b_gat, a_src, a_dst, W1, b1, W2, b2,
           Wf, bf):
    src = edge_index[0]
    dst = edge_index[1]
    eye = jnp.repeat(jnp.eye(HEADS, dtype=_f32), HID, axis=0)
    pad = jnp.zeros((HEADS * HID, 8), _f32)
    As = jnp.concatenate([a_src.reshape(-1)[:, None] * eye, pad], axis=1)
    Ad = jnp.concatenate([a_dst.reshape(-1)[:, None] * eye, pad], axis=1)

    h0, h1, ast, adt = _tc1(x, W_gat, As, Ad)

    z16 = jnp.zeros((RPT, 16), _f32)
    z128 = jnp.zeros((RPT, 128), _f32)
    ex, den0, den1 = _sc_a(src, dst, ast, adt, z16)
    alpha = _sc_c(dst, ex, den0, den1)
    a00, a01, a10, a11 = _sc_d(src, dst, h0, h1, alpha, z128)
    g0, g1 = _tc2(a00, a10, a01, a11, b_gat.reshape(1, HEADS * HID))
    n00, n01, n10, n11 = _sc_e(src, dst, g0, g1, z128)
    emb = _tc3(g0, g1, n00, n01, n10, n11, batch.reshape(N, 1),
               W1[:128], W1[128:], b1.reshape(1, HID), W2, b2.reshape(1, HID),
               Wf, bf.reshape(1, OUT))
    return emb


# speedup vs baseline: 19.9007x; 19.9007x over previous
import functools

import jax
import jax.numpy as jnp
from jax import lax
from jax.experimental import pallas as pl
from jax.experimental.pallas import tpu as pltpu
from jax.experimental.pallas import tpu_sc as plsc

N = 10000
E = 320000
IN = 128
HID = 32
HEADS = 8
OUT = 128
G = 64

NC = 2
NS = 16
NW = NC * NS
EPW = E // NW
CH = 80
NCHUNK = EPW // CH
NPAD = 10240
RPT = NPAD // NS
NV16 = EPW // 16
RB = 2000
GRID = N // RB

_f32 = jnp.float32


def _tc1_body(x_ref, wg_ref, as_ref, ad_ref, h0_ref, h1_ref, ts_ref, td_ref):
    h = jnp.dot(x_ref[...], wg_ref[...])
    h0_ref[...] = h[:, :128]
    h1_ref[...] = h[:, 128:]
    ts_ref[...] = jnp.dot(h, as_ref[...])
    td_ref[...] = jnp.dot(h, ad_ref[...])


def _tc1(x, W_gat, As, Ad):
    return pl.pallas_call(
        _tc1_body,
        grid=(GRID,),
        in_specs=[
            pl.BlockSpec((RB, IN), lambda i: (i, 0)),
            pl.BlockSpec((IN, HEADS * HID), lambda i: (0, 0)),
            pl.BlockSpec((HEADS * HID, HEADS), lambda i: (0, 0)),
            pl.BlockSpec((HEADS * HID, HEADS), lambda i: (0, 0)),
        ],
        out_specs=[
            pl.BlockSpec((RB, 128), lambda i: (i, 0)),
            pl.BlockSpec((RB, 128), lambda i: (i, 0)),
            pl.BlockSpec((RB, HEADS), lambda i: (i, 0)),
            pl.BlockSpec((RB, HEADS), lambda i: (i, 0)),
        ],
        out_shape=[
            jax.ShapeDtypeStruct((N, 128), _f32),
            jax.ShapeDtypeStruct((N, 128), _f32),
            jax.ShapeDtypeStruct((N, HEADS), _f32),
            jax.ShapeDtypeStruct((N, HEADS), _f32),
        ],
    )(x, W_gat, As, Ad)


def _tc2_body(a00_ref, a10_ref, a01_ref, a11_ref, b_ref, g0_ref, g1_ref):
    b = b_ref[...]
    g0_ref[...] = jnp.maximum(a00_ref[...] + a10_ref[...] + b[:, :128], 0.0)
    g1_ref[...] = jnp.maximum(a01_ref[...] + a11_ref[...] + b[:, 128:], 0.0)


def _tc2(a00, a10, a01, a11, b2d):
    return pl.pallas_call(
        _tc2_body,
        grid=(GRID,),
        in_specs=[pl.BlockSpec((RB, 128), lambda i: (i, 0))] * 4
        + [pl.BlockSpec((1, HEADS * HID), lambda i: (0, 0))],
        out_specs=[pl.BlockSpec((RB, 128), lambda i: (i, 0))] * 2,
        out_shape=[jax.ShapeDtypeStruct((N, 128), _f32)] * 2,
    )(a00, a10, a01, a11, b2d)


def _tc3_body(g0_ref, g1_ref, n00_ref, n01_ref, n10_ref, n11_ref, bt_ref,
              w1a_ref, w1b_ref, b1_ref, w2_ref, b2_ref, wf_ref, bf_ref,
              emb_ref, sums, counts):
    i = pl.program_id(0)

    @pl.when(i == 0)
    def _():
        sums[...] = jnp.zeros_like(sums)
        counts[...] = jnp.zeros_like(counts)

    gin0 = g0_ref[...] + n00_ref[...] + n10_ref[...]
    gin1 = g1_ref[...] + n01_ref[...] + n11_ref[...]
    h1 = jnp.maximum(
        jnp.dot(gin0, w1a_ref[...]) + jnp.dot(gin1, w1b_ref[...]) + b1_ref[...], 0.0)
    hg = jnp.maximum(jnp.dot(h1, w2_ref[...]) + b2_ref[...], 0.0)
    ids = jnp.broadcast_to(bt_ref[...], (RB, G))
    iota = lax.broadcasted_iota(jnp.int32, (RB, G), 1)
    onehot = (ids == iota).astype(_f32)
    sums[...] += lax.dot_general(onehot, hg, (((0,), (0,)), ((), ())))
    counts[...] += lax.dot_general(
        onehot, jnp.ones((RB, HID), _f32), (((0,), (0,)), ((), ())))

    @pl.when(i == GRID - 1)
    def _():
        pooled = sums[...] / jnp.maximum(counts[...], 1.0)
        emb_ref[...] = jnp.dot(pooled, wf_ref[...]) + bf_ref[...]


def _tc3(g0, g1, n00, n01, n10, n11, bt, W1a, W1b, b1, W2, b2, Wf, bf):
    return pl.pallas_call(
        _tc3_body,
        grid=(GRID,),
        in_specs=[pl.BlockSpec((RB, 128), lambda i: (i, 0))] * 6
        + [
            pl.BlockSpec((RB, 1), lambda i: (i, 0)),
            pl.BlockSpec((128, HID), lambda i: (0, 0)),
            pl.BlockSpec((128, HID), lambda i: (0, 0)),
            pl.BlockSpec((1, HID), lambda i: (0, 0)),
            pl.BlockSpec((HID, HID), lambda i: (0, 0)),
            pl.BlockSpec((1, HID), lambda i: (0, 0)),
            pl.BlockSpec((HID, OUT), lambda i: (0, 0)),
            pl.BlockSpec((1, OUT), lambda i: (0, 0)),
        ],
        out_specs=pl.BlockSpec((G, OUT), lambda i: (0, 0)),
        out_shape=jax.ShapeDtypeStruct((G, OUT), _f32),
        scratch_shapes=[pltpu.VMEM((G, HID), _f32), pltpu.VMEM((G, HID), _f32)],
    )(g0, g1, n00, n01, n10, n11, bt, W1a, W1b, b1, W2, b2, Wf, bf)


_MESH = plsc.VectorSubcoreMesh(core_axis_name="c", subcore_axis_name="s")


def _sc_a(src, dst, asf, adf, znp):
    @functools.partial(
        pl.kernel, mesh=_MESH,
        compiler_params=pltpu.CompilerParams(needs_layout_passes=False),
        out_type=[
            jax.ShapeDtypeStruct((HEADS * E,), _f32),
            jax.ShapeDtypeStruct((NC * HEADS * NPAD,), _f32),
        ],
        scratch_types=[
            pltpu.VMEM((EPW,), jnp.int32),
            pltpu.VMEM((EPW,), jnp.int32),
            pltpu.VMEM((N,), _f32),
            pltpu.VMEM((N,), _f32),
            pltpu.VMEM((NPAD,), _f32),
            pltpu.VMEM((EPW,), _f32),
            pltpu.VMEM((RPT,), _f32),
            pltpu.VMEM((RPT,), _f32),
            pltpu.VMEM_SHARED((NS * NPAD,), _f32),
        ],
    )
    def k(src_h, dst_h, asf_h, adf_h, znp_h, ex_h, den_h,
          src_v, dst_v, ta, tb, den_v, ex_own, acc_v, tmp_v, red_sp):
        c = lax.axis_index("c")
        s = lax.axis_index("s")
        base = (c * NS + s) * EPW
        pltpu.sync_copy(src_h.at[pl.ds(base, EPW)], src_v)
        pltpu.sync_copy(dst_h.at[pl.ds(base, EPW)], dst_v)
        for h in range(HEADS):
            pltpu.sync_copy(asf_h.at[pl.ds(h * N, N)], ta)
            pltpu.sync_copy(adf_h.at[pl.ds(h * N, N)], tb)
            pltpu.sync_copy(znp_h, den_v)

            def vec(j, carry):
                o = pl.multiple_of(j * 16, 8)
                s16 = src_v[pl.ds(o, 16)]
                d16 = dst_v[pl.ds(o, 16)]
                va = plsc.load_gather(ta, [s16])
                vb = plsc.load_gather(tb, [d16])
                v = va + vb
                ex16 = jnp.exp(jnp.maximum(v, 0.2 * v))
                ex_own[pl.ds(o, 16)] = ex16
                plsc.addupdate_scatter(den_v, [d16], ex16)
                return carry

            lax.fori_loop(0, NV16, vec, 0)
            pltpu.sync_copy(ex_own, ex_h.at[pl.ds(h * E + base, EPW)])
            pltpu.sync_copy(den_v, red_sp.at[pl.ds(s * NPAD, NPAD)])
            plsc.subcore_barrier()
            pltpu.sync_copy(red_sp.at[pl.ds(s * RPT, RPT)], acc_v)
            for t in range(1, NS):
                pltpu.sync_copy(
                    red_sp.at[pl.ds(t * NPAD + s * RPT, RPT)], tmp_v)

                def radd(j, carry):
                    o = pl.multiple_of(j * 16, 8)
                    acc_v[pl.ds(o, 16)] = acc_v[pl.ds(o, 16)] + tmp_v[pl.ds(o, 16)]
                    return carry

                lax.fori_loop(0, RPT // 16, radd, 0)
            pltpu.sync_copy(
                acc_v, den_h.at[pl.ds((c * HEADS + h) * NPAD + s * RPT, RPT)])
            plsc.subcore_barrier()

    return k(src, dst, asf, adf, znp)


def _sc_c(dst, ex, den):
    @functools.partial(
        pl.kernel, mesh=_MESH,
        compiler_params=pltpu.CompilerParams(needs_layout_passes=False),
        out_type=jax.ShapeDtypeStruct((HEADS * E,), _f32),
        scratch_types=[
            pltpu.VMEM((EPW,), jnp.int32),
            pltpu.VMEM((NPAD,), _f32),
            pltpu.VMEM((NPAD,), _f32),
            pltpu.VMEM((EPW,), _f32),
        ],
    )
    def k(dst_h, ex_h, den_h, al_h, dst_v, d0, d1, ev):
        c = lax.axis_index("c")
        s = lax.axis_index("s")
        base = (c * NS + s) * EPW
        pltpu.sync_copy(dst_h.at[pl.ds(base, EPW)], dst_v)
        for h in range(HEADS):
            pltpu.sync_copy(den_h.at[pl.ds(h * NPAD, NPAD)], d0)
            pltpu.sync_copy(den_h.at[pl.ds((HEADS + h) * NPAD, NPAD)], d1)
            pltpu.sync_copy(ex_h.at[pl.ds(h * E + base, EPW)], ev)

            def vec(j, carry):
                o = pl.multiple_of(j * 16, 8)
                d16 = dst_v[pl.ds(o, 16)]
                v0 = plsc.load_gather(d0, [d16])
                v1 = plsc.load_gather(d1, [d16])
                ev[pl.ds(o, 16)] = ev[pl.ds(o, 16)] / (v0 + v1 + 1e-16)
                return carry

            lax.fori_loop(0, NV16, vec, 0)
            pltpu.sync_copy(ev, al_h.at[pl.ds(h * E + base, EPW)])

    return k(dst, ex, den)


def _sc_d(src, dst, h0, h1, alpha, z128):
    @functools.partial(
        pl.kernel, mesh=_MESH,
        out_type=[jax.ShapeDtypeStruct((NPAD, 128), _f32)] * 4,
        scratch_types=[
            pltpu.VMEM((CH,), jnp.int32),
            pltpu.VMEM((CH,), jnp.int32),
            pltpu.VMEM((CH,), _f32),
            pltpu.VMEM((CH,), _f32),
            pltpu.VMEM((CH,), _f32),
            pltpu.VMEM((CH,), _f32),
            pltpu.VMEM((CH, 128), _f32),
            pltpu.VMEM_SHARED((NPAD, 128), _f32),
            pltpu.SemaphoreType.DMA,
        ],
    )
    def k(src_h, dst_h, h0_h, h1_h, al_h, z128_h,
          a00_h, a01_h, a10_h, a11_h,
          src_v, dst_v, av0, av1, av2, av3, hrows, agg_sp, sem1):
        c = lax.axis_index("c")
        s = lax.axis_index("s")
        base = (c * NS + s) * EPW
        rs = pl.ds(s * RPT, RPT)
        avs = (av0, av1, av2, av3)
        outs = ((a00_h, a01_h), (a10_h, a11_h))
        for g in range(2):
            htab = h0_h if g == 0 else h1_h
            pltpu.sync_copy(z128_h, agg_sp.at[rs])
            plsc.subcore_barrier()

            def chunk(i, carry, g=g, htab=htab):
                off = pl.multiple_of(base + i * CH, 8)
                pltpu.sync_copy(src_h.at[pl.ds(off, CH)], src_v)
                pltpu.sync_copy(dst_h.at[pl.ds(off, CH)], dst_v)
                for h4 in range(4):
                    pltpu.sync_copy(
                        al_h.at[pl.ds((4 * g + h4) * E + off, CH)], avs[h4])
                pltpu.async_copy(htab.at[src_v], hrows, sem1).wait()

                def grp(gi, carry2):
                    o = pl.multiple_of(gi * 16, 8)
                    a16 = [avs[h4][pl.ds(o, 16)] for h4 in range(4)]
                    for j in range(16):
                        e = o + j
                        for h4 in range(4):
                            a = a16[h4][j]
                            for b in range(2):
                                sl = pl.ds(h4 * 32 + b * 16, 16)
                                hrows[e, sl] = hrows[e, sl] * a
                    return carry2

                lax.fori_loop(0, CH // 16, grp, 0)
                pltpu.sync_copy(hrows, agg_sp.at[dst_v], add=True)
                return carry

            lax.fori_loop(0, NCHUNK, chunk, 0)
            plsc.subcore_barrier()

            @pl.when(c == 0)
            def _(g=g):
                pltpu.sync_copy(agg_sp.at[rs], outs[0][g].at[rs])

            @pl.when(c == 1)
            def _(g=g):
                pltpu.sync_copy(agg_sp.at[rs], outs[1][g].at[rs])

    return k(src, dst, h0, h1, alpha, z128)


def _sc_e(src, dst, g0, g1, z128):
    @functools.partial(
        pl.kernel, mesh=_MESH,
        out_type=[jax.ShapeDtypeStruct((NPAD, 128), _f32)] * 4,
        scratch_types=[
            pltpu.VMEM((CH,), jnp.int32),
            pltpu.VMEM((CH,), jnp.int32),
            pltpu.VMEM((CH, 128), _f32),
            pltpu.VMEM_SHARED((NPAD, 128), _f32),
            pltpu.SemaphoreType.DMA,
        ],
    )
    def k(src_h, dst_h, g0_h, g1_h, z128_h,
          n00_h, n01_h, n10_h, n11_h,
          src_v, dst_v, grows, nb_sp, sem1):
        c = lax.axis_index("c")
        s = lax.axis_index("s")
        base = (c * NS + s) * EPW
        rs = pl.ds(s * RPT, RPT)
        outs = ((n00_h, n01_h), (n10_h, n11_h))
        for g in range(2):
            gtab = g0_h if g == 0 else g1_h
            pltpu.sync_copy(z128_h, nb_sp.at[rs])
            plsc.subcore_barrier()

            def chunk(i, carry, gtab=gtab):
                off = pl.multiple_of(base + i * CH, 8)
                pltpu.sync_copy(src_h.at[pl.ds(off, CH)], src_v)
                pltpu.sync_copy(dst_h.at[pl.ds(off, CH)], dst_v)
                pltpu.async_copy(gtab.at[src_v], grows, sem1).wait()
                pltpu.sync_copy(grows, nb_sp.at[dst_v], add=True)
                return carry

            lax.fori_loop(0, NCHUNK, chunk, 0)
            plsc.subcore_barrier()

            @pl.when(c == 0)
            def _(g=g):
                pltpu.sync_copy(nb_sp.at[rs], outs[0][g].at[rs])

            @pl.when(c == 1)
            def _(g=g):
                pltpu.sync_copy(nb_sp.at[rs], outs[1][g].at[rs])

    return k(src, dst, g0, g1, z128)


def kernel(x, edge_index, batch, W_gat, b_gat, a_src, a_dst, W1, b1, W2, b2,
           Wf, bf):
    src = edge_index[0]
    dst = edge_index[1]
    eye = jnp.repeat(jnp.eye(HEADS, dtype=_f32), HID, axis=0)
    As = a_src.reshape(-1)[:, None] * eye
    Ad = a_dst.reshape(-1)[:, None] * eye

    h0, h1, ts, td = _tc1(x, W_gat, As, Ad)
    asf = ts.T.reshape(HEADS * N)
    adf = td.T.reshape(HEADS * N)

    znp = jnp.zeros((NPAD,), _f32)
    z128 = jnp.zeros((RPT, 128), _f32)
    ex, den = _sc_a(src, dst, asf, adf, znp)
    alpha = _sc_c(dst, ex, den)
    a00, a01, a10, a11 = _sc_d(src, dst, h0, h1, alpha, z128)
    g0, g1 = _tc2(a00, a10, a01, a11, b_gat.reshape(1, HEADS * HID))
    n00, n01, n10, n11 = _sc_e(src, dst, g0, g1, z128)
    emb = _tc3(g0, g1, n00, n01, n10, n11, batch.reshape(N, 1),
               W1[:128], W1[128:], b1.reshape(1, HID), W2, b2.reshape(1, HID),
               Wf, bf.reshape(1, OUT))
    return emb


# speedup vs baseline: 26.8338x; 1.3484x over previous
import functools

import jax
import jax.numpy as jnp
from jax import lax
from jax.experimental import pallas as pl
from jax.experimental.pallas import tpu as pltpu
from jax.experimental.pallas import tpu_sc as plsc

N = 10000
E = 320000
IN = 128
HID = 32
HEADS = 8
OUT = 128
G = 64

NC = 2
NS = 16
NW = NC * NS
EPW = E // NW
CH = 80
NCHUNK = EPW // CH
NPAD = 10240
RPT = NPAD // NS
NV16 = EPW // 16
RB = 2000
GRID = N // RB

_f32 = jnp.float32


def _tc1_body(x_ref, wg_ref, as_ref, ad_ref, h0_ref, h1_ref, ts_ref, td_ref):
    h = jnp.dot(x_ref[...], wg_ref[...])
    h0_ref[...] = h[:, :128]
    h1_ref[...] = h[:, 128:]
    ts_ref[...] = jnp.dot(h, as_ref[...])
    td_ref[...] = jnp.dot(h, ad_ref[...])


def _tc1(x, W_gat, As, Ad):
    return pl.pallas_call(
        _tc1_body,
        grid=(GRID,),
        in_specs=[
            pl.BlockSpec((RB, IN), lambda i: (i, 0)),
            pl.BlockSpec((IN, HEADS * HID), lambda i: (0, 0)),
            pl.BlockSpec((HEADS * HID, HEADS), lambda i: (0, 0)),
            pl.BlockSpec((HEADS * HID, HEADS), lambda i: (0, 0)),
        ],
        out_specs=[
            pl.BlockSpec((RB, 128), lambda i: (i, 0)),
            pl.BlockSpec((RB, 128), lambda i: (i, 0)),
            pl.BlockSpec((RB, HEADS), lambda i: (i, 0)),
            pl.BlockSpec((RB, HEADS), lambda i: (i, 0)),
        ],
        out_shape=[
            jax.ShapeDtypeStruct((N, 128), _f32),
            jax.ShapeDtypeStruct((N, 128), _f32),
            jax.ShapeDtypeStruct((N, HEADS), _f32),
            jax.ShapeDtypeStruct((N, HEADS), _f32),
        ],
    )(x, W_gat, As, Ad)


def _tc2_body(a00_ref, a10_ref, a01_ref, a11_ref, b_ref, g0_ref, g1_ref):
    b = b_ref[...]
    g0_ref[...] = jnp.maximum(a00_ref[...] + a10_ref[...] + b[:, :128], 0.0)
    g1_ref[...] = jnp.maximum(a01_ref[...] + a11_ref[...] + b[:, 128:], 0.0)


def _tc2(a00, a10, a01, a11, b2d):
    return pl.pallas_call(
        _tc2_body,
        grid=(GRID,),
        in_specs=[pl.BlockSpec((RB, 128), lambda i: (i, 0))] * 4
        + [pl.BlockSpec((1, HEADS * HID), lambda i: (0, 0))],
        out_specs=[pl.BlockSpec((RB, 128), lambda i: (i, 0))] * 2,
        out_shape=[jax.ShapeDtypeStruct((N, 128), _f32)] * 2,
    )(a00, a10, a01, a11, b2d)


def _tc3_body(g0_ref, g1_ref, n00_ref, n01_ref, n10_ref, n11_ref, bt_ref,
              w1a_ref, w1b_ref, b1_ref, w2_ref, b2_ref, wf_ref, bf_ref,
              emb_ref, sums, counts):
    i = pl.program_id(0)

    @pl.when(i == 0)
    def _():
        sums[...] = jnp.zeros_like(sums)
        counts[...] = jnp.zeros_like(counts)

    gin0 = g0_ref[...] + n00_ref[...] + n10_ref[...]
    gin1 = g1_ref[...] + n01_ref[...] + n11_ref[...]
    h1 = jnp.maximum(
        jnp.dot(gin0, w1a_ref[...]) + jnp.dot(gin1, w1b_ref[...]) + b1_ref[...], 0.0)
    hg = jnp.maximum(jnp.dot(h1, w2_ref[...]) + b2_ref[...], 0.0)
    ids = jnp.broadcast_to(bt_ref[...], (RB, G))
    iota = lax.broadcasted_iota(jnp.int32, (RB, G), 1)
    onehot = (ids == iota).astype(_f32)
    sums[...] += lax.dot_general(onehot, hg, (((0,), (0,)), ((), ())))
    counts[...] += lax.dot_general(
        onehot, jnp.ones((RB, HID), _f32), (((0,), (0,)), ((), ())))

    @pl.when(i == GRID - 1)
    def _():
        pooled = sums[...] / jnp.maximum(counts[...], 1.0)
        emb_ref[...] = jnp.dot(pooled, wf_ref[...]) + bf_ref[...]


def _tc3(g0, g1, n00, n01, n10, n11, bt, W1a, W1b, b1, W2, b2, Wf, bf):
    return pl.pallas_call(
        _tc3_body,
        grid=(GRID,),
        in_specs=[pl.BlockSpec((RB, 128), lambda i: (i, 0))] * 6
        + [
            pl.BlockSpec((RB, 1), lambda i: (i, 0)),
            pl.BlockSpec((128, HID), lambda i: (0, 0)),
            pl.BlockSpec((128, HID), lambda i: (0, 0)),
            pl.BlockSpec((1, HID), lambda i: (0, 0)),
            pl.BlockSpec((HID, HID), lambda i: (0, 0)),
            pl.BlockSpec((1, HID), lambda i: (0, 0)),
            pl.BlockSpec((HID, OUT), lambda i: (0, 0)),
            pl.BlockSpec((1, OUT), lambda i: (0, 0)),
        ],
        out_specs=pl.BlockSpec((G, OUT), lambda i: (0, 0)),
        out_shape=jax.ShapeDtypeStruct((G, OUT), _f32),
        scratch_shapes=[pltpu.VMEM((G, HID), _f32), pltpu.VMEM((G, HID), _f32)],
    )(g0, g1, n00, n01, n10, n11, bt, W1a, W1b, b1, W2, b2, Wf, bf)


_MESH = plsc.VectorSubcoreMesh(core_axis_name="c", subcore_axis_name="s")


def _sc_a(src, dst, asf, adf, znp):
    @functools.partial(
        pl.kernel, mesh=_MESH,
        compiler_params=pltpu.CompilerParams(needs_layout_passes=False),
        out_type=[
            jax.ShapeDtypeStruct((HEADS * E,), _f32),
            jax.ShapeDtypeStruct((NC * HEADS * NPAD,), _f32),
        ],
        scratch_types=[
            pltpu.VMEM((EPW,), jnp.int32),
            pltpu.VMEM((EPW,), jnp.int32),
            pltpu.VMEM((N,), _f32),
            pltpu.VMEM((N,), _f32),
            pltpu.VMEM((NPAD,), _f32),
            pltpu.VMEM((EPW,), _f32),
            pltpu.VMEM((RPT,), _f32),
            pltpu.VMEM((RPT,), _f32),
            pltpu.VMEM_SHARED((NS * NPAD,), _f32),
        ],
    )
    def k(src_h, dst_h, asf_h, adf_h, znp_h, ex_h, den_h,
          src_v, dst_v, ta, tb, den_v, ex_own, acc_v, tmp_v, red_sp):
        c = lax.axis_index("c")
        s = lax.axis_index("s")
        base = (c * NS + s) * EPW
        pltpu.sync_copy(src_h.at[pl.ds(base, EPW)], src_v)
        pltpu.sync_copy(dst_h.at[pl.ds(base, EPW)], dst_v)
        for h in range(HEADS):
            pltpu.sync_copy(asf_h.at[pl.ds(h * N, N)], ta)
            pltpu.sync_copy(adf_h.at[pl.ds(h * N, N)], tb)
            pltpu.sync_copy(znp_h, den_v)

            def vec(j, carry):
                o = pl.multiple_of(j * 16, 8)
                s16 = src_v[pl.ds(o, 16)]
                d16 = dst_v[pl.ds(o, 16)]
                va = plsc.load_gather(ta, [s16])
                vb = plsc.load_gather(tb, [d16])
                v = va + vb
                ex16 = jnp.exp(jnp.maximum(v, 0.2 * v))
                ex_own[pl.ds(o, 16)] = ex16
                plsc.addupdate_scatter(den_v, [d16], ex16)
                return carry

            lax.fori_loop(0, NV16, vec, 0)
            pltpu.sync_copy(ex_own, ex_h.at[pl.ds(h * E + base, EPW)])
            pltpu.sync_copy(den_v, red_sp.at[pl.ds(s * NPAD, NPAD)])
            plsc.subcore_barrier()
            pltpu.sync_copy(red_sp.at[pl.ds(s * RPT, RPT)], acc_v)
            for t in range(1, NS):
                pltpu.sync_copy(
                    red_sp.at[pl.ds(t * NPAD + s * RPT, RPT)], tmp_v)

                def radd(j, carry):
                    o = pl.multiple_of(j * 16, 8)
                    acc_v[pl.ds(o, 16)] = acc_v[pl.ds(o, 16)] + tmp_v[pl.ds(o, 16)]
                    return carry

                lax.fori_loop(0, RPT // 16, radd, 0)
            pltpu.sync_copy(
                acc_v, den_h.at[pl.ds((c * HEADS + h) * NPAD + s * RPT, RPT)])
            plsc.subcore_barrier()

    return k(src, dst, asf, adf, znp)


def _sc_c(dst, ex, den):
    @functools.partial(
        pl.kernel, mesh=_MESH,
        compiler_params=pltpu.CompilerParams(needs_layout_passes=False),
        out_type=jax.ShapeDtypeStruct((HEADS * E,), _f32),
        scratch_types=[
            pltpu.VMEM((EPW,), jnp.int32),
            pltpu.VMEM((NPAD,), _f32),
            pltpu.VMEM((NPAD,), _f32),
            pltpu.VMEM((EPW,), _f32),
        ],
    )
    def k(dst_h, ex_h, den_h, al_h, dst_v, d0, d1, ev):
        c = lax.axis_index("c")
        s = lax.axis_index("s")
        base = (c * NS + s) * EPW
        pltpu.sync_copy(dst_h.at[pl.ds(base, EPW)], dst_v)
        for h in range(HEADS):
            pltpu.sync_copy(den_h.at[pl.ds(h * NPAD, NPAD)], d0)
            pltpu.sync_copy(den_h.at[pl.ds((HEADS + h) * NPAD, NPAD)], d1)
            pltpu.sync_copy(ex_h.at[pl.ds(h * E + base, EPW)], ev)

            def vec(j, carry):
                o = pl.multiple_of(j * 16, 8)
                d16 = dst_v[pl.ds(o, 16)]
                v0 = plsc.load_gather(d0, [d16])
                v1 = plsc.load_gather(d1, [d16])
                ev[pl.ds(o, 16)] = ev[pl.ds(o, 16)] / (v0 + v1 + 1e-16)
                return carry

            lax.fori_loop(0, NV16, vec, 0)
            pltpu.sync_copy(ev, al_h.at[pl.ds(h * E + base, EPW)])

    return k(dst, ex, den)


def _sc_d(src_e, dst_e, h0, h1, alpha, z128):
    @functools.partial(
        pl.kernel, mesh=_MESH,
        out_type=[jax.ShapeDtypeStruct((NPAD, 128), _f32)] * 4,
        scratch_types=[
            pltpu.VMEM((CH,), jnp.int32),
            pltpu.VMEM((CH,), jnp.int32),
            pltpu.VMEM((CH,), jnp.int32),
            pltpu.VMEM((CH,), jnp.int32),
            pltpu.VMEM((CH,), _f32),
            pltpu.VMEM((CH,), _f32),
            pltpu.VMEM((CH,), _f32),
            pltpu.VMEM((CH,), _f32),
            pltpu.VMEM((CH, 128), _f32),
            pltpu.VMEM((CH, 128), _f32),
            pltpu.VMEM_SHARED((NPAD, 128), _f32),
            pltpu.SemaphoreType.DMA,
            pltpu.SemaphoreType.DMA,
            pltpu.SemaphoreType.DMA,
            pltpu.SemaphoreType.DMA,
        ],
    )
    def k(src_h, dst_h, h0_h, h1_h, al_h, z128_h,
          a00_h, a01_h, a10_h, a11_h,
          si0, si1, di0, di1, av0, av1, av2, av3, hb0, hb1, agg_sp,
          sg0, sg1, ss0, ss1):
        c = lax.axis_index("c")
        s = lax.axis_index("s")
        base = (c * NS + s) * EPW
        rs = pl.ds(s * RPT, RPT)
        sis = (si0, si1)
        dis = (di0, di1)
        avs = (av0, av1, av2, av3)
        hbs = (hb0, hb1)
        sgs = (sg0, sg1)
        sss = (ss0, ss1)
        outs = ((a00_h, a01_h), (a10_h, a11_h))
        for g in range(2):
            htab = h0_h if g == 0 else h1_h
            pltpu.sync_copy(z128_h, agg_sp.at[rs])
            plsc.subcore_barrier()

            def startG(ci, b, htab=htab):
                off = pl.multiple_of(base + ci * CH, 8)
                pltpu.sync_copy(src_h.at[pl.ds(off, CH)], sis[b])
                pltpu.async_copy(htab.at[sis[b]], hbs[b], sgs[b])

            def waitG(b, htab=htab):
                pltpu.make_async_copy(htab.at[sis[b]], hbs[b], sgs[b]).wait()

            def startS(ci, b):
                off = pl.multiple_of(base + ci * CH, 8)
                pltpu.sync_copy(dst_h.at[pl.ds(off, CH)], dis[b])
                return pltpu.async_copy(
                    hbs[b], agg_sp.at[dis[b]], sss[b], add=True)

            def scale(ci, b, g=g):
                hb = hbs[b]
                off = pl.multiple_of(base + ci * CH, 8)
                for h4 in range(4):
                    pltpu.sync_copy(
                        al_h.at[pl.ds((4 * g + h4) * E + off, CH)], avs[h4])

                def grp(gi, carry):
                    o = pl.multiple_of(gi * 16, 8)
                    a16 = [avs[h4][pl.ds(o, 16)] for h4 in range(4)]
                    for j in range(16):
                        e = o + j
                        for h4 in range(4):
                            a = a16[h4][j]
                            for bb in range(2):
                                sl = pl.ds(h4 * 32 + bb * 16, 16)
                                hb[e, sl] = hb[e, sl] * a
                    return carry

                lax.fori_loop(0, CH // 16, grp, 0)

            startG(0, 0)
            startG(1, 1)

            def pair(i2, carry):
                cc = i2 * 2
                waitG(0)
                scale(cc, 0)
                cp0 = startS(cc, 0)
                waitG(1)
                scale(cc + 1, 1)
                cp1 = startS(cc + 1, 1)
                cp0.wait()
                startG(cc + 2, 0)
                cp1.wait()

                @pl.when(cc + 3 < NCHUNK)
                def _():
                    startG(cc + 3, 1)

                return carry

            lax.fori_loop(0, (NCHUNK - 1) // 2, pair, 0)
            waitG(0)
            scale(NCHUNK - 1, 0)
            startS(NCHUNK - 1, 0).wait()
            plsc.subcore_barrier()

            @pl.when(c == 0)
            def _(g=g):
                pltpu.sync_copy(agg_sp.at[rs], outs[0][g].at[rs])

            @pl.when(c == 1)
            def _(g=g):
                pltpu.sync_copy(agg_sp.at[rs], outs[1][g].at[rs])

    return k(src_e, dst_e, h0, h1, alpha, z128)


def _sc_e(src_e, dst_e, g0, g1, z128):
    @functools.partial(
        pl.kernel, mesh=_MESH,
        out_type=[jax.ShapeDtypeStruct((NPAD, 128), _f32)] * 4,
        scratch_types=[
            pltpu.VMEM((CH,), jnp.int32),
            pltpu.VMEM((CH,), jnp.int32),
            pltpu.VMEM((CH,), jnp.int32),
            pltpu.VMEM((CH,), jnp.int32),
            pltpu.VMEM((CH, 128), _f32),
            pltpu.VMEM((CH, 128), _f32),
            pltpu.VMEM_SHARED((NPAD, 128), _f32),
            pltpu.SemaphoreType.DMA,
            pltpu.SemaphoreType.DMA,
            pltpu.SemaphoreType.DMA,
            pltpu.SemaphoreType.DMA,
        ],
    )
    def k(src_h, dst_h, g0_h, g1_h, z128_h,
          n00_h, n01_h, n10_h, n11_h,
          si0, si1, di0, di1, hb0, hb1, nb_sp, sg0, sg1, ss0, ss1):
        c = lax.axis_index("c")
        s = lax.axis_index("s")
        base = (c * NS + s) * EPW
        rs = pl.ds(s * RPT, RPT)
        sis = (si0, si1)
        dis = (di0, di1)
        hbs = (hb0, hb1)
        sgs = (sg0, sg1)
        sss = (ss0, ss1)
        outs = ((n00_h, n01_h), (n10_h, n11_h))
        for g in range(2):
            gtab = g0_h if g == 0 else g1_h
            pltpu.sync_copy(z128_h, nb_sp.at[rs])
            plsc.subcore_barrier()

            def startG(ci, b, gtab=gtab):
                off = pl.multiple_of(base + ci * CH, 8)
                pltpu.sync_copy(src_h.at[pl.ds(off, CH)], sis[b])
                pltpu.async_copy(gtab.at[sis[b]], hbs[b], sgs[b])

            def waitG(b, gtab=gtab):
                pltpu.make_async_copy(gtab.at[sis[b]], hbs[b], sgs[b]).wait()

            def startS(ci, b):
                off = pl.multiple_of(base + ci * CH, 8)
                pltpu.sync_copy(dst_h.at[pl.ds(off, CH)], dis[b])
                return pltpu.async_copy(
                    hbs[b], nb_sp.at[dis[b]], sss[b], add=True)

            startG(0, 0)
            startG(1, 1)

            def pair(i2, carry):
                cc = i2 * 2
                waitG(0)
                cp0 = startS(cc, 0)
                waitG(1)
                cp1 = startS(cc + 1, 1)
                cp0.wait()
                startG(cc + 2, 0)
                cp1.wait()

                @pl.when(cc + 3 < NCHUNK)
                def _():
                    startG(cc + 3, 1)

                return carry

            lax.fori_loop(0, (NCHUNK - 1) // 2, pair, 0)
            waitG(0)
            startS(NCHUNK - 1, 0).wait()
            plsc.subcore_barrier()

            @pl.when(c == 0)
            def _(g=g):
                pltpu.sync_copy(nb_sp.at[rs], outs[0][g].at[rs])

            @pl.when(c == 1)
            def _(g=g):
                pltpu.sync_copy(nb_sp.at[rs], outs[1][g].at[rs])

    return k(src_e, dst_e, g0, g1, z128)


def kernel(x, edge_index, batch, W_gat, b_gat, a_src, a_dst, W1, b1, W2, b2,
           Wf, bf):
    src = edge_index[0]
    dst = edge_index[1]
    eye = jnp.repeat(jnp.eye(HEADS, dtype=_f32), HID, axis=0)
    As = a_src.reshape(-1)[:, None] * eye
    Ad = a_dst.reshape(-1)[:, None] * eye

    h0, h1, ts, td = _tc1(x, W_gat, As, Ad)
    asf = ts.T.reshape(HEADS * N)
    adf = td.T.reshape(HEADS * N)

    znp = jnp.zeros((NPAD,), _f32)
    z128 = jnp.zeros((RPT, 128), _f32)
    ex, den = _sc_a(src, dst, asf, adf, znp)
    alpha = _sc_c(dst, ex, den)
    a00, a01, a10, a11 = _sc_d(src, dst, h0, h1, alpha, z128)
    g0, g1 = _tc2(a00, a10, a01, a11, b_gat.reshape(1, HEADS * HID))
    n00, n01, n10, n11 = _sc_e(src, dst, g0, g1, z128)
    emb = _tc3(g0, g1, n00, n01, n10, n11, batch.reshape(N, 1),
               W1[:128], W1[128:], b1.reshape(1, HID), W2, b2.reshape(1, HID),
               Wf, bf.reshape(1, OUT))
    return emb


# speedup vs baseline: 27.7781x; 1.0352x over previous
import functools

import jax
import jax.numpy as jnp
from jax import lax
from jax.experimental import pallas as pl
from jax.experimental.pallas import tpu as pltpu
from jax.experimental.pallas import tpu_sc as plsc

N = 10000
E = 320000
IN = 128
HID = 32
HEADS = 8
OUT = 128
G = 64

NC = 2
NS = 16
NW = NC * NS
EPW = E // NW
CH = 80
NCHUNK = EPW // CH
NPAD = 10240
RPT = NPAD // NS
NV16 = EPW // 16
RB = 2000
GRID = N // RB

_f32 = jnp.float32


def _tc1_body(x_ref, wg_ref, as_ref, ad_ref, h0_ref, h1_ref, ts_ref, td_ref):
    h = jnp.dot(x_ref[...], wg_ref[...])
    h0_ref[...] = h[:, :128]
    h1_ref[...] = h[:, 128:]
    ts_ref[...] = jnp.dot(h, as_ref[...])
    td_ref[...] = jnp.dot(h, ad_ref[...])


def _tc1(x, W_gat, As, Ad):
    return pl.pallas_call(
        _tc1_body,
        grid=(GRID,),
        in_specs=[
            pl.BlockSpec((RB, IN), lambda i: (i, 0)),
            pl.BlockSpec((IN, HEADS * HID), lambda i: (0, 0)),
            pl.BlockSpec((HEADS * HID, HEADS), lambda i: (0, 0)),
            pl.BlockSpec((HEADS * HID, HEADS), lambda i: (0, 0)),
        ],
        out_specs=[
            pl.BlockSpec((RB, 128), lambda i: (i, 0)),
            pl.BlockSpec((RB, 128), lambda i: (i, 0)),
            pl.BlockSpec((RB, HEADS), lambda i: (i, 0)),
            pl.BlockSpec((RB, HEADS), lambda i: (i, 0)),
        ],
        out_shape=[
            jax.ShapeDtypeStruct((N, 128), _f32),
            jax.ShapeDtypeStruct((N, 128), _f32),
            jax.ShapeDtypeStruct((N, HEADS), _f32),
            jax.ShapeDtypeStruct((N, HEADS), _f32),
        ],
    )(x, W_gat, As, Ad)


def _tc2_body(a00_ref, a10_ref, a01_ref, a11_ref, b_ref, g0_ref, g1_ref):
    b = b_ref[...]
    g0_ref[...] = jnp.maximum(a00_ref[...] + a10_ref[...] + b[:, :128], 0.0)
    g1_ref[...] = jnp.maximum(a01_ref[...] + a11_ref[...] + b[:, 128:], 0.0)


def _tc2(a00, a10, a01, a11, b2d):
    return pl.pallas_call(
        _tc2_body,
        grid=(GRID,),
        in_specs=[pl.BlockSpec((RB, 128), lambda i: (i, 0))] * 4
        + [pl.BlockSpec((1, HEADS * HID), lambda i: (0, 0))],
        out_specs=[pl.BlockSpec((RB, 128), lambda i: (i, 0))] * 2,
        out_shape=[jax.ShapeDtypeStruct((N, 128), _f32)] * 2,
    )(a00, a10, a01, a11, b2d)


def _tc3_body(g0_ref, g1_ref, n00_ref, n01_ref, n10_ref, n11_ref, bt_ref,
              w1a_ref, w1b_ref, b1_ref, w2_ref, b2_ref, wf_ref, bf_ref,
              emb_ref, sums, counts):
    i = pl.program_id(0)

    @pl.when(i == 0)
    def _():
        sums[...] = jnp.zeros_like(sums)
        counts[...] = jnp.zeros_like(counts)

    gin0 = g0_ref[...] + n00_ref[...] + n10_ref[...]
    gin1 = g1_ref[...] + n01_ref[...] + n11_ref[...]
    h1 = jnp.maximum(
        jnp.dot(gin0, w1a_ref[...]) + jnp.dot(gin1, w1b_ref[...]) + b1_ref[...], 0.0)
    hg = jnp.maximum(jnp.dot(h1, w2_ref[...]) + b2_ref[...], 0.0)
    ids = jnp.broadcast_to(bt_ref[...], (RB, G))
    iota = lax.broadcasted_iota(jnp.int32, (RB, G), 1)
    onehot = (ids == iota).astype(_f32)
    sums[...] += lax.dot_general(onehot, hg, (((0,), (0,)), ((), ())))
    counts[...] += lax.dot_general(
        onehot, jnp.ones((RB, HID), _f32), (((0,), (0,)), ((), ())))

    @pl.when(i == GRID - 1)
    def _():
        pooled = sums[...] / jnp.maximum(counts[...], 1.0)
        emb_ref[...] = jnp.dot(pooled, wf_ref[...]) + bf_ref[...]


def _tc3(g0, g1, n00, n01, n10, n11, bt, W1a, W1b, b1, W2, b2, Wf, bf):
    return pl.pallas_call(
        _tc3_body,
        grid=(GRID,),
        in_specs=[pl.BlockSpec((RB, 128), lambda i: (i, 0))] * 6
        + [
            pl.BlockSpec((RB, 1), lambda i: (i, 0)),
            pl.BlockSpec((128, HID), lambda i: (0, 0)),
            pl.BlockSpec((128, HID), lambda i: (0, 0)),
            pl.BlockSpec((1, HID), lambda i: (0, 0)),
            pl.BlockSpec((HID, HID), lambda i: (0, 0)),
            pl.BlockSpec((1, HID), lambda i: (0, 0)),
            pl.BlockSpec((HID, OUT), lambda i: (0, 0)),
            pl.BlockSpec((1, OUT), lambda i: (0, 0)),
        ],
        out_specs=pl.BlockSpec((G, OUT), lambda i: (0, 0)),
        out_shape=jax.ShapeDtypeStruct((G, OUT), _f32),
        scratch_shapes=[pltpu.VMEM((G, HID), _f32), pltpu.VMEM((G, HID), _f32)],
    )(g0, g1, n00, n01, n10, n11, bt, W1a, W1b, b1, W2, b2, Wf, bf)


_MESH = plsc.VectorSubcoreMesh(core_axis_name="c", subcore_axis_name="s")


def _sc_a(src, dst, asf, adf, znp):
    @functools.partial(
        pl.kernel, mesh=_MESH,
        compiler_params=pltpu.CompilerParams(needs_layout_passes=False),
        out_type=[
            jax.ShapeDtypeStruct((HEADS * E,), _f32),
            jax.ShapeDtypeStruct((NC * HEADS * NPAD,), _f32),
        ],
        scratch_types=[
            pltpu.VMEM((EPW,), jnp.int32),
            pltpu.VMEM((EPW,), jnp.int32),
            pltpu.VMEM((N,), _f32),
            pltpu.VMEM((N,), _f32),
            pltpu.VMEM((NPAD,), _f32),
            pltpu.VMEM((EPW,), _f32),
            pltpu.VMEM((RPT,), _f32),
            pltpu.VMEM(((NS - 1) * RPT,), _f32),
            pltpu.VMEM_SHARED((NS * NPAD,), _f32),
            pltpu.SemaphoreType.DMA,
            pltpu.SemaphoreType.DMA,
            pltpu.SemaphoreType.DMA,
            pltpu.SemaphoreType.DMA,
            pltpu.SemaphoreType.DMA,
            pltpu.SemaphoreType.DMA,
            pltpu.SemaphoreType.DMA,
        ],
    )
    def k(src_h, dst_h, asf_h, adf_h, znp_h, ex_h, den_h,
          src_v, dst_v, ta, tb, den_v, ex_own, acc_v, tmpbig, red_sp,
          sma, smb, smz, sme, sms, smr, smo):
        c = lax.axis_index("c")
        s = lax.axis_index("s")
        base = (c * NS + s) * EPW
        pltpu.sync_copy(src_h.at[pl.ds(base, EPW)], src_v)
        pltpu.sync_copy(dst_h.at[pl.ds(base, EPW)], dst_v)
        for h in range(HEADS):
            cpa = pltpu.async_copy(asf_h.at[pl.ds(h * N, N)], ta, sma)
            cpb = pltpu.async_copy(adf_h.at[pl.ds(h * N, N)], tb, smb)
            cpz = pltpu.async_copy(znp_h, den_v, smz)
            cpa.wait()
            cpb.wait()
            cpz.wait()

            def vec(j, carry):
                o = pl.multiple_of(j * 16, 8)
                s16 = src_v[pl.ds(o, 16)]
                d16 = dst_v[pl.ds(o, 16)]
                va = plsc.load_gather(ta, [s16])
                vb = plsc.load_gather(tb, [d16])
                v = va + vb
                ex16 = jnp.exp(jnp.maximum(v, 0.2 * v))
                ex_own[pl.ds(o, 16)] = ex16
                plsc.addupdate_scatter(den_v, [d16], ex16)
                return carry

            lax.fori_loop(0, NV16, vec, 0)
            cpe = pltpu.async_copy(ex_own, ex_h.at[pl.ds(h * E + base, EPW)], sme)
            pltpu.async_copy(den_v, red_sp.at[pl.ds(s * NPAD, NPAD)], sms).wait()
            plsc.subcore_barrier()
            cps = []
            for t in range(NS - 1):
                oslot = lax.rem(s + 1 + t, NS) * NPAD + s * RPT
                cps.append(pltpu.async_copy(
                    red_sp.at[pl.ds(oslot, RPT)],
                    tmpbig.at[pl.ds(t * RPT, RPT)], smr))
            for cp in cps:
                cp.wait()

            def radd(j, carry):
                o = pl.multiple_of(j * 16, 8)
                v = den_v[pl.ds(s * RPT + o, 16)]
                for t in range(NS - 1):
                    v = v + tmpbig[pl.ds(t * RPT + o, 16)]
                acc_v[pl.ds(o, 16)] = v
                return carry

            lax.fori_loop(0, RPT // 16, radd, 0)
            cpo = pltpu.async_copy(
                acc_v, den_h.at[pl.ds((c * HEADS + h) * NPAD + s * RPT, RPT)],
                smo)
            cpe.wait()
            cpo.wait()
            plsc.subcore_barrier()

    return k(src, dst, asf, adf, znp)


def _sc_c(dst, ex, den):
    @functools.partial(
        pl.kernel, mesh=_MESH,
        compiler_params=pltpu.CompilerParams(needs_layout_passes=False),
        out_type=jax.ShapeDtypeStruct((HEADS * E,), _f32),
        scratch_types=[
            pltpu.VMEM((EPW,), jnp.int32),
            pltpu.VMEM((NPAD,), _f32),
            pltpu.VMEM((NPAD,), _f32),
            pltpu.VMEM((EPW,), _f32),
            pltpu.SemaphoreType.DMA,
            pltpu.SemaphoreType.DMA,
            pltpu.SemaphoreType.DMA,
            pltpu.SemaphoreType.DMA,
        ],
    )
    def k(dst_h, ex_h, den_h, al_h, dst_v, d0, d1, ev, sm0, sm1, sm2, smw):
        c = lax.axis_index("c")
        s = lax.axis_index("s")
        base = (c * NS + s) * EPW
        pltpu.sync_copy(dst_h.at[pl.ds(base, EPW)], dst_v)
        for h in range(HEADS):
            cp0 = pltpu.async_copy(den_h.at[pl.ds(h * NPAD, NPAD)], d0, sm0)
            cp1 = pltpu.async_copy(
                den_h.at[pl.ds((HEADS + h) * NPAD, NPAD)], d1, sm1)
            cp2 = pltpu.async_copy(ex_h.at[pl.ds(h * E + base, EPW)], ev, sm2)
            cp0.wait()
            cp1.wait()
            cp2.wait()

            def vec(j, carry):
                o = pl.multiple_of(j * 16, 8)
                d16 = dst_v[pl.ds(o, 16)]
                v0 = plsc.load_gather(d0, [d16])
                v1 = plsc.load_gather(d1, [d16])
                ev[pl.ds(o, 16)] = ev[pl.ds(o, 16)] / (v0 + v1 + 1e-16)
                return carry

            lax.fori_loop(0, NV16, vec, 0)
            pltpu.async_copy(ev, al_h.at[pl.ds(h * E + base, EPW)], smw).wait()

    return k(dst, ex, den)


def _sc_d(src_e, dst_e, h0, h1, alpha, z128):
    @functools.partial(
        pl.kernel, mesh=_MESH,
        out_type=[jax.ShapeDtypeStruct((NPAD, 128), _f32)] * 4,
        scratch_types=[
            pltpu.VMEM((CH,), jnp.int32),
            pltpu.VMEM((CH,), jnp.int32),
            pltpu.VMEM((CH,), jnp.int32),
            pltpu.VMEM((CH,), jnp.int32),
            pltpu.VMEM((CH,), _f32),
            pltpu.VMEM((CH,), _f32),
            pltpu.VMEM((CH,), _f32),
            pltpu.VMEM((CH,), _f32),
            pltpu.VMEM((CH, 128), _f32),
            pltpu.VMEM((CH, 128), _f32),
            pltpu.VMEM_SHARED((NPAD, 128), _f32),
            pltpu.SemaphoreType.DMA,
            pltpu.SemaphoreType.DMA,
            pltpu.SemaphoreType.DMA,
            pltpu.SemaphoreType.DMA,
        ],
    )
    def k(src_h, dst_h, h0_h, h1_h, al_h, z128_h,
          a00_h, a01_h, a10_h, a11_h,
          si0, si1, di0, di1, av0, av1, av2, av3, hb0, hb1, agg_sp,
          sg0, sg1, ss0, ss1):
        c = lax.axis_index("c")
        s = lax.axis_index("s")
        base = (c * NS + s) * EPW
        rs = pl.ds(s * RPT, RPT)
        sis = (si0, si1)
        dis = (di0, di1)
        avs = (av0, av1, av2, av3)
        hbs = (hb0, hb1)
        sgs = (sg0, sg1)
        sss = (ss0, ss1)
        outs = ((a00_h, a01_h), (a10_h, a11_h))
        for g in range(2):
            htab = h0_h if g == 0 else h1_h
            pltpu.sync_copy(z128_h, agg_sp.at[rs])
            plsc.subcore_barrier()

            def startG(ci, b, htab=htab):
                off = pl.multiple_of(base + ci * CH, 8)
                pltpu.sync_copy(src_h.at[pl.ds(off, CH)], sis[b])
                pltpu.async_copy(htab.at[sis[b]], hbs[b], sgs[b])

            def waitG(b, htab=htab):
                pltpu.make_async_copy(htab.at[sis[b]], hbs[b], sgs[b]).wait()

            def startS(ci, b):
                off = pl.multiple_of(base + ci * CH, 8)
                pltpu.sync_copy(dst_h.at[pl.ds(off, CH)], dis[b])
                return pltpu.async_copy(
                    hbs[b], agg_sp.at[dis[b]], sss[b], add=True)

            def scale(ci, b, g=g):
                hb = hbs[b]
                off = pl.multiple_of(base + ci * CH, 8)
                for h4 in range(4):
                    pltpu.sync_copy(
                        al_h.at[pl.ds((4 * g + h4) * E + off, CH)], avs[h4])

                def grp(gi, carry):
                    o = pl.multiple_of(gi * 16, 8)
                    a16 = [avs[h4][pl.ds(o, 16)] for h4 in range(4)]
                    for j in range(16):
                        e = o + j
                        for h4 in range(4):
                            a = a16[h4][j]
                            for bb in range(2):
                                sl = pl.ds(h4 * 32 + bb * 16, 16)
                                hb[e, sl] = hb[e, sl] * a
                    return carry

                lax.fori_loop(0, CH // 16, grp, 0)

            startG(0, 0)
            startG(1, 1)

            def pair(i2, carry):
                cc = i2 * 2
                waitG(0)
                scale(cc, 0)
                cp0 = startS(cc, 0)
                waitG(1)
                scale(cc + 1, 1)
                cp1 = startS(cc + 1, 1)
                cp0.wait()
                startG(cc + 2, 0)
                cp1.wait()

                @pl.when(cc + 3 < NCHUNK)
                def _():
                    startG(cc + 3, 1)

                return carry

            lax.fori_loop(0, (NCHUNK - 1) // 2, pair, 0)
            waitG(0)
            scale(NCHUNK - 1, 0)
            startS(NCHUNK - 1, 0).wait()
            plsc.subcore_barrier()

            @pl.when(c == 0)
            def _(g=g):
                pltpu.sync_copy(agg_sp.at[rs], outs[0][g].at[rs])

            @pl.when(c == 1)
            def _(g=g):
                pltpu.sync_copy(agg_sp.at[rs], outs[1][g].at[rs])

    return k(src_e, dst_e, h0, h1, alpha, z128)


def _sc_e(src_e, dst_e, g0, g1, z128):
    @functools.partial(
        pl.kernel, mesh=_MESH,
        out_type=[jax.ShapeDtypeStruct((NPAD, 128), _f32)] * 4,
        scratch_types=[
            pltpu.VMEM((CH,), jnp.int32),
            pltpu.VMEM((CH,), jnp.int32),
            pltpu.VMEM((CH,), jnp.int32),
            pltpu.VMEM((CH,), jnp.int32),
            pltpu.VMEM((CH, 128), _f32),
            pltpu.VMEM((CH, 128), _f32),
            pltpu.VMEM_SHARED((NPAD, 128), _f32),
            pltpu.SemaphoreType.DMA,
            pltpu.SemaphoreType.DMA,
            pltpu.SemaphoreType.DMA,
            pltpu.SemaphoreType.DMA,
        ],
    )
    def k(src_h, dst_h, g0_h, g1_h, z128_h,
          n00_h, n01_h, n10_h, n11_h,
          si0, si1, di0, di1, hb0, hb1, nb_sp, sg0, sg1, ss0, ss1):
        c = lax.axis_index("c")
        s = lax.axis_index("s")
        base = (c * NS + s) * EPW
        rs = pl.ds(s * RPT, RPT)
        sis = (si0, si1)
        dis = (di0, di1)
        hbs = (hb0, hb1)
        sgs = (sg0, sg1)
        sss = (ss0, ss1)
        outs = ((n00_h, n01_h), (n10_h, n11_h))
        for g in range(2):
            gtab = g0_h if g == 0 else g1_h
            pltpu.sync_copy(z128_h, nb_sp.at[rs])
            plsc.subcore_barrier()

            def startG(ci, b, gtab=gtab):
                off = pl.multiple_of(base + ci * CH, 8)
                pltpu.sync_copy(src_h.at[pl.ds(off, CH)], sis[b])
                pltpu.async_copy(gtab.at[sis[b]], hbs[b], sgs[b])

            def waitG(b, gtab=gtab):
                pltpu.make_async_copy(gtab.at[sis[b]], hbs[b], sgs[b]).wait()

            def startS(ci, b):
                off = pl.multiple_of(base + ci * CH, 8)
                pltpu.sync_copy(dst_h.at[pl.ds(off, CH)], dis[b])
                return pltpu.async_copy(
                    hbs[b], nb_sp.at[dis[b]], sss[b], add=True)

            startG(0, 0)
            startG(1, 1)

            def pair(i2, carry):
                cc = i2 * 2
                waitG(0)
                cp0 = startS(cc, 0)
                waitG(1)
                cp1 = startS(cc + 1, 1)
                cp0.wait()
                startG(cc + 2, 0)
                cp1.wait()

                @pl.when(cc + 3 < NCHUNK)
                def _():
                    startG(cc + 3, 1)

                return carry

            lax.fori_loop(0, (NCHUNK - 1) // 2, pair, 0)
            waitG(0)
            startS(NCHUNK - 1, 0).wait()
            plsc.subcore_barrier()

            @pl.when(c == 0)
            def _(g=g):
                pltpu.sync_copy(nb_sp.at[rs], outs[0][g].at[rs])

            @pl.when(c == 1)
            def _(g=g):
                pltpu.sync_copy(nb_sp.at[rs], outs[1][g].at[rs])

    return k(src_e, dst_e, g0, g1, z128)


def kernel(x, edge_index, batch, W_gat, b_gat, a_src, a_dst, W1, b1, W2, b2,
           Wf, bf):
    src = edge_index[0]
    dst = edge_index[1]
    eye = jnp.repeat(jnp.eye(HEADS, dtype=_f32), HID, axis=0)
    As = a_src.reshape(-1)[:, None] * eye
    Ad = a_dst.reshape(-1)[:, None] * eye

    h0, h1, ts, td = _tc1(x, W_gat, As, Ad)
    asf = ts.T.reshape(HEADS * N)
    adf = td.T.reshape(HEADS * N)

    znp = jnp.zeros((NPAD,), _f32)
    z128 = jnp.zeros((RPT, 128), _f32)
    ex, den = _sc_a(src, dst, asf, adf, znp)
    alpha = _sc_c(dst, ex, den)
    a00, a01, a10, a11 = _sc_d(src, dst, h0, h1, alpha, z128)
    g0, g1 = _tc2(a00, a10, a01, a11, b_gat.reshape(1, HEADS * HID))
    n00, n01, n10, n11 = _sc_e(src, dst, g0, g1, z128)
    emb = _tc3(g0, g1, n00, n01, n10, n11, batch.reshape(N, 1),
               W1[:128], W1[128:], b1.reshape(1, HID), W2, b2.reshape(1, HID),
               Wf, bf.reshape(1, OUT))
    return emb


# speedup vs baseline: 38.0688x; 1.3705x over previous
import functools

import jax
import jax.numpy as jnp
from jax import lax
from jax.experimental import pallas as pl
from jax.experimental.pallas import tpu as pltpu
from jax.experimental.pallas import tpu_sc as plsc

N = 10000
E = 320000
IN = 128
HID = 32
HEADS = 8
OUT = 128
G = 64

NC = 2
NS = 16
NW = NC * NS
EPW = E // NW
CH = 80
NCHUNK = EPW // CH
NPAD = 10240
RPT = NPAD // NS
NV16 = EPW // 16
RB = 2000
GRID = N // RB

_f32 = jnp.float32


def _tc1_body(x_ref, wg_ref, as_ref, ad_ref, h0_ref, h1_ref, ts_ref, td_ref):
    h = jnp.dot(x_ref[...], wg_ref[...])
    h0_ref[...] = h[:, :128]
    h1_ref[...] = h[:, 128:]
    ts_ref[...] = jnp.dot(h, as_ref[...])
    td_ref[...] = jnp.dot(h, ad_ref[...])


def _tc1(x, W_gat, As, Ad):
    return pl.pallas_call(
        _tc1_body,
        grid=(GRID,),
        in_specs=[
            pl.BlockSpec((RB, IN), lambda i: (i, 0)),
            pl.BlockSpec((IN, HEADS * HID), lambda i: (0, 0)),
            pl.BlockSpec((HEADS * HID, HEADS), lambda i: (0, 0)),
            pl.BlockSpec((HEADS * HID, HEADS), lambda i: (0, 0)),
        ],
        out_specs=[
            pl.BlockSpec((RB, 128), lambda i: (i, 0)),
            pl.BlockSpec((RB, 128), lambda i: (i, 0)),
            pl.BlockSpec((RB, HEADS), lambda i: (i, 0)),
            pl.BlockSpec((RB, HEADS), lambda i: (i, 0)),
        ],
        out_shape=[
            jax.ShapeDtypeStruct((N, 128), _f32),
            jax.ShapeDtypeStruct((N, 128), _f32),
            jax.ShapeDtypeStruct((N, HEADS), _f32),
            jax.ShapeDtypeStruct((N, HEADS), _f32),
        ],
    )(x, W_gat, As, Ad)


def _tc2_body(a00_ref, a10_ref, a01_ref, a11_ref, b_ref, g0_ref, g1_ref):
    b = b_ref[...]
    g0_ref[...] = jnp.maximum(a00_ref[...] + a10_ref[...] + b[:, :128], 0.0)
    g1_ref[...] = jnp.maximum(a01_ref[...] + a11_ref[...] + b[:, 128:], 0.0)


def _tc2(a00, a10, a01, a11, b2d):
    return pl.pallas_call(
        _tc2_body,
        grid=(GRID,),
        in_specs=[pl.BlockSpec((RB, 128), lambda i: (i, 0))] * 4
        + [pl.BlockSpec((1, HEADS * HID), lambda i: (0, 0))],
        out_specs=[pl.BlockSpec((RB, 128), lambda i: (i, 0))] * 2,
        out_shape=[jax.ShapeDtypeStruct((N, 128), _f32)] * 2,
    )(a00, a10, a01, a11, b2d)


def _tc3_body(g0_ref, g1_ref, n00_ref, n01_ref, n10_ref, n11_ref, bt_ref,
              w1a_ref, w1b_ref, b1_ref, w2_ref, b2_ref, wf_ref, bf_ref,
              emb_ref, sums, counts):
    i = pl.program_id(0)

    @pl.when(i == 0)
    def _():
        sums[...] = jnp.zeros_like(sums)
        counts[...] = jnp.zeros_like(counts)

    gin0 = g0_ref[...] + n00_ref[...] + n10_ref[...]
    gin1 = g1_ref[...] + n01_ref[...] + n11_ref[...]
    h1 = jnp.maximum(
        jnp.dot(gin0, w1a_ref[...]) + jnp.dot(gin1, w1b_ref[...]) + b1_ref[...], 0.0)
    hg = jnp.maximum(jnp.dot(h1, w2_ref[...]) + b2_ref[...], 0.0)
    ids = jnp.broadcast_to(bt_ref[...], (RB, G))
    iota = lax.broadcasted_iota(jnp.int32, (RB, G), 1)
    onehot = (ids == iota).astype(_f32)
    sums[...] += lax.dot_general(onehot, hg, (((0,), (0,)), ((), ())))
    counts[...] += lax.dot_general(
        onehot, jnp.ones((RB, HID), _f32), (((0,), (0,)), ((), ())))

    @pl.when(i == GRID - 1)
    def _():
        pooled = sums[...] / jnp.maximum(counts[...], 1.0)
        emb_ref[...] = jnp.dot(pooled, wf_ref[...]) + bf_ref[...]


def _tc3(g0, g1, n00, n01, n10, n11, bt, W1a, W1b, b1, W2, b2, Wf, bf):
    return pl.pallas_call(
        _tc3_body,
        grid=(GRID,),
        in_specs=[pl.BlockSpec((RB, 128), lambda i: (i, 0))] * 6
        + [
            pl.BlockSpec((RB, 1), lambda i: (i, 0)),
            pl.BlockSpec((128, HID), lambda i: (0, 0)),
            pl.BlockSpec((128, HID), lambda i: (0, 0)),
            pl.BlockSpec((1, HID), lambda i: (0, 0)),
            pl.BlockSpec((HID, HID), lambda i: (0, 0)),
            pl.BlockSpec((1, HID), lambda i: (0, 0)),
            pl.BlockSpec((HID, OUT), lambda i: (0, 0)),
            pl.BlockSpec((1, OUT), lambda i: (0, 0)),
        ],
        out_specs=pl.BlockSpec((G, OUT), lambda i: (0, 0)),
        out_shape=jax.ShapeDtypeStruct((G, OUT), _f32),
        scratch_shapes=[pltpu.VMEM((G, HID), _f32), pltpu.VMEM((G, HID), _f32)],
    )(g0, g1, n00, n01, n10, n11, bt, W1a, W1b, b1, W2, b2, Wf, bf)


_MESH = plsc.VectorSubcoreMesh(core_axis_name="c", subcore_axis_name="s")


def _sc_a(src, dst, asf, adf, znp):
    @functools.partial(
        pl.kernel, mesh=_MESH,
        compiler_params=pltpu.CompilerParams(needs_layout_passes=False),
        out_type=[
            jax.ShapeDtypeStruct((HEADS * E,), _f32),
            jax.ShapeDtypeStruct((NC * HEADS * NPAD,), _f32),
        ],
        scratch_types=[
            pltpu.VMEM((EPW,), jnp.int32),
            pltpu.VMEM((EPW,), jnp.int32),
            pltpu.VMEM((N,), _f32),
            pltpu.VMEM((N,), _f32),
            pltpu.VMEM((NPAD,), _f32),
            pltpu.VMEM((EPW,), _f32),
            pltpu.VMEM((RPT,), _f32),
            pltpu.VMEM(((NS - 1) * RPT,), _f32),
            pltpu.VMEM_SHARED((NS * NPAD,), _f32),
            pltpu.SemaphoreType.DMA,
            pltpu.SemaphoreType.DMA,
            pltpu.SemaphoreType.DMA,
            pltpu.SemaphoreType.DMA,
            pltpu.SemaphoreType.DMA,
            pltpu.SemaphoreType.DMA,
            pltpu.SemaphoreType.DMA,
        ],
    )
    def k(src_h, dst_h, asf_h, adf_h, znp_h, ex_h, den_h,
          src_v, dst_v, ta, tb, den_v, ex_own, acc_v, tmpbig, red_sp,
          sma, smb, smz, sme, sms, smr, smo):
        c = lax.axis_index("c")
        s = lax.axis_index("s")
        base = (c * NS + s) * EPW
        pltpu.sync_copy(src_h.at[pl.ds(base, EPW)], src_v)
        pltpu.sync_copy(dst_h.at[pl.ds(base, EPW)], dst_v)
        for h in range(HEADS):
            cpa = pltpu.async_copy(asf_h.at[pl.ds(h * N, N)], ta, sma)
            cpb = pltpu.async_copy(adf_h.at[pl.ds(h * N, N)], tb, smb)
            cpz = pltpu.async_copy(znp_h, den_v, smz)
            cpa.wait()
            cpb.wait()
            cpz.wait()

            def vec(j, carry):
                o = pl.multiple_of(j * 16, 8)
                s16 = src_v[pl.ds(o, 16)]
                d16 = dst_v[pl.ds(o, 16)]
                va = plsc.load_gather(ta, [s16])
                vb = plsc.load_gather(tb, [d16])
                v = va + vb
                ex16 = jnp.exp(jnp.maximum(v, 0.2 * v))
                ex_own[pl.ds(o, 16)] = ex16
                plsc.addupdate_scatter(den_v, [d16], ex16)
                return carry

            lax.fori_loop(0, NV16, vec, 0)
            cpe = pltpu.async_copy(ex_own, ex_h.at[pl.ds(h * E + base, EPW)], sme)
            pltpu.async_copy(den_v, red_sp.at[pl.ds(s * NPAD, NPAD)], sms).wait()
            plsc.subcore_barrier()
            cps = []
            for t in range(NS - 1):
                oslot = lax.rem(s + 1 + t, NS) * NPAD + s * RPT
                cps.append(pltpu.async_copy(
                    red_sp.at[pl.ds(oslot, RPT)],
                    tmpbig.at[pl.ds(t * RPT, RPT)], smr))
            for cp in cps:
                cp.wait()

            def radd(j, carry):
                o = pl.multiple_of(j * 16, 8)
                v = den_v[pl.ds(s * RPT + o, 16)]
                for t in range(NS - 1):
                    v = v + tmpbig[pl.ds(t * RPT + o, 16)]
                acc_v[pl.ds(o, 16)] = v
                return carry

            lax.fori_loop(0, RPT // 16, radd, 0)
            cpo = pltpu.async_copy(
                acc_v, den_h.at[pl.ds((c * HEADS + h) * NPAD + s * RPT, RPT)],
                smo)
            cpe.wait()
            cpo.wait()
            plsc.subcore_barrier()

    return k(src, dst, asf, adf, znp)


def _sc_c(dst, ex, den):
    @functools.partial(
        pl.kernel, mesh=_MESH,
        compiler_params=pltpu.CompilerParams(needs_layout_passes=False),
        out_type=jax.ShapeDtypeStruct((HEADS * E,), _f32),
        scratch_types=[
            pltpu.VMEM((EPW,), jnp.int32),
            pltpu.VMEM((NPAD,), _f32),
            pltpu.VMEM((NPAD,), _f32),
            pltpu.VMEM((EPW,), _f32),
            pltpu.SemaphoreType.DMA,
            pltpu.SemaphoreType.DMA,
            pltpu.SemaphoreType.DMA,
            pltpu.SemaphoreType.DMA,
        ],
    )
    def k(dst_h, ex_h, den_h, al_h, dst_v, d0, d1, ev, sm0, sm1, sm2, smw):
        c = lax.axis_index("c")
        s = lax.axis_index("s")
        base = (c * NS + s) * EPW
        pltpu.sync_copy(dst_h.at[pl.ds(base, EPW)], dst_v)
        for h in range(HEADS):
            cp0 = pltpu.async_copy(den_h.at[pl.ds(h * NPAD, NPAD)], d0, sm0)
            cp1 = pltpu.async_copy(
                den_h.at[pl.ds((HEADS + h) * NPAD, NPAD)], d1, sm1)
            cp2 = pltpu.async_copy(ex_h.at[pl.ds(h * E + base, EPW)], ev, sm2)
            cp0.wait()
            cp1.wait()
            cp2.wait()

            def vec(j, carry):
                o = pl.multiple_of(j * 16, 8)
                d16 = dst_v[pl.ds(o, 16)]
                v0 = plsc.load_gather(d0, [d16])
                v1 = plsc.load_gather(d1, [d16])
                ev[pl.ds(o, 16)] = ev[pl.ds(o, 16)] / (v0 + v1 + 1e-16)
                return carry

            lax.fori_loop(0, NV16, vec, 0)
            pltpu.async_copy(ev, al_h.at[pl.ds(h * E + base, EPW)], smw).wait()

    return k(dst, ex, den)


def _sc_d(src_e, dst_e, h0, h1, alpha, z128):
    @functools.partial(
        pl.kernel, mesh=_MESH,
        out_type=[jax.ShapeDtypeStruct((NPAD, 128), _f32)] * 4,
        scratch_types=[
            pltpu.VMEM((CH,), jnp.int32),
            pltpu.VMEM((CH,), jnp.int32),
            pltpu.VMEM((CH,), jnp.int32),
            pltpu.VMEM((CH,), jnp.int32),
            pltpu.VMEM((CH,), _f32),
            pltpu.VMEM((CH,), _f32),
            pltpu.VMEM((CH,), _f32),
            pltpu.VMEM((CH,), _f32),
            pltpu.VMEM((CH,), _f32),
            pltpu.VMEM((CH,), _f32),
            pltpu.VMEM((CH,), _f32),
            pltpu.VMEM((CH,), _f32),
            pltpu.VMEM((CH, 128), _f32),
            pltpu.VMEM((CH, 128), _f32),
            pltpu.VMEM_SHARED((NPAD, 128), _f32),
            pltpu.SemaphoreType.DMA,
            pltpu.SemaphoreType.DMA,
            pltpu.SemaphoreType.DMA,
            pltpu.SemaphoreType.DMA,
        ],
    )
    def k(src_h, dst_h, h0_h, h1_h, al_h, z128_h,
          a00_h, a01_h, a10_h, a11_h,
          si0, si1, di0, di1, av00, av01, av02, av03,
          av10, av11, av12, av13, hb0, hb1, agg_sp,
          sg0, sg1, ss0, ss1):
        c = lax.axis_index("c")
        s = lax.axis_index("s")
        base = (c * NS + s) * EPW
        rs = pl.ds(s * RPT, RPT)
        sis = (si0, si1)
        dis = (di0, di1)
        avs = ((av00, av01, av02, av03), (av10, av11, av12, av13))
        hbs = (hb0, hb1)
        sgs = (sg0, sg1)
        sss = (ss0, ss1)
        outs = ((a00_h, a01_h), (a10_h, a11_h))
        for g in range(2):
            htab = h0_h if g == 0 else h1_h
            pltpu.sync_copy(z128_h, agg_sp.at[rs])
            plsc.subcore_barrier()

            def startG(ci, b, g=g, htab=htab):
                off = pl.multiple_of(base + ci * CH, 8)
                pltpu.sync_copy(src_h.at[pl.ds(off, CH)], sis[b])
                pltpu.async_copy(htab.at[sis[b]], hbs[b], sgs[b])
                for h4 in range(4):
                    pltpu.async_copy(
                        al_h.at[pl.ds((4 * g + h4) * E + off, CH)],
                        avs[b][h4], sgs[b])

            def waitG(b, g=g, htab=htab):
                pltpu.make_async_copy(htab.at[sis[b]], hbs[b], sgs[b]).wait()
                for h4 in range(4):
                    pltpu.make_async_copy(
                        al_h.at[pl.ds(base, CH)], avs[b][h4], sgs[b]).wait()

            def startS(ci, b):
                off = pl.multiple_of(base + ci * CH, 8)
                pltpu.sync_copy(dst_h.at[pl.ds(off, CH)], dis[b])
                return pltpu.async_copy(
                    hbs[b], agg_sp.at[dis[b]], sss[b], add=True)

            def scale(b):
                hb = hbs[b]
                av = avs[b]

                def grp(gi, carry):
                    o = pl.multiple_of(gi * 16, 8)
                    a16 = [av[h4][pl.ds(o, 16)] for h4 in range(4)]
                    for j in range(16):
                        e = o + j
                        for h4 in range(4):
                            a = a16[h4][j]
                            for bb in range(2):
                                sl = pl.ds(h4 * 32 + bb * 16, 16)
                                hb[e, sl] = hb[e, sl] * a
                    return carry

                lax.fori_loop(0, CH // 16, grp, 0)

            startG(0, 0)
            startG(1, 1)

            def pair(i2, carry):
                cc = i2 * 2
                waitG(0)
                scale(0)
                cp0 = startS(cc, 0)
                waitG(1)
                scale(1)
                cp1 = startS(cc + 1, 1)
                cp0.wait()
                startG(cc + 2, 0)
                cp1.wait()

                @pl.when(cc + 3 < NCHUNK)
                def _():
                    startG(cc + 3, 1)

                return carry

            lax.fori_loop(0, (NCHUNK - 1) // 2, pair, 0)
            waitG(0)
            scale(0)
            startS(NCHUNK - 1, 0).wait()
            plsc.subcore_barrier()

            @pl.when(c == 0)
            def _(g=g):
                pltpu.sync_copy(agg_sp.at[rs], outs[0][g].at[rs])

            @pl.when(c == 1)
            def _(g=g):
                pltpu.sync_copy(agg_sp.at[rs], outs[1][g].at[rs])

    return k(src_e, dst_e, h0, h1, alpha, z128)


def _sc_e(src_e, dst_e, g0, g1, z128):
    @functools.partial(
        pl.kernel, mesh=_MESH,
        out_type=[jax.ShapeDtypeStruct((NPAD, 128), _f32)] * 4,
        scratch_types=[
            pltpu.VMEM((CH,), jnp.int32),
            pltpu.VMEM((CH,), jnp.int32),
            pltpu.VMEM((CH,), jnp.int32),
            pltpu.VMEM((CH,), jnp.int32),
            pltpu.VMEM((CH, 128), _f32),
            pltpu.VMEM((CH, 128), _f32),
            pltpu.VMEM_SHARED((NPAD, 128), _f32),
            pltpu.SemaphoreType.DMA,
            pltpu.SemaphoreType.DMA,
            pltpu.SemaphoreType.DMA,
            pltpu.SemaphoreType.DMA,
        ],
    )
    def k(src_h, dst_h, g0_h, g1_h, z128_h,
          n00_h, n01_h, n10_h, n11_h,
          si0, si1, di0, di1, hb0, hb1, nb_sp, sg0, sg1, ss0, ss1):
        c = lax.axis_index("c")
        s = lax.axis_index("s")
        base = (c * NS + s) * EPW
        rs = pl.ds(s * RPT, RPT)
        sis = (si0, si1)
        dis = (di0, di1)
        hbs = (hb0, hb1)
        sgs = (sg0, sg1)
        sss = (ss0, ss1)
        outs = ((n00_h, n01_h), (n10_h, n11_h))
        for g in range(2):
            gtab = g0_h if g == 0 else g1_h
            pltpu.sync_copy(z128_h, nb_sp.at[rs])
            plsc.subcore_barrier()

            def startG(ci, b, gtab=gtab):
                off = pl.multiple_of(base + ci * CH, 8)
                pltpu.sync_copy(src_h.at[pl.ds(off, CH)], sis[b])
                pltpu.async_copy(gtab.at[sis[b]], hbs[b], sgs[b])

            def waitG(b, gtab=gtab):
                pltpu.make_async_copy(gtab.at[sis[b]], hbs[b], sgs[b]).wait()

            def startS(ci, b):
                off = pl.multiple_of(base + ci * CH, 8)
                pltpu.sync_copy(dst_h.at[pl.ds(off, CH)], dis[b])
                return pltpu.async_copy(
                    hbs[b], nb_sp.at[dis[b]], sss[b], add=True)

            startG(0, 0)
            startG(1, 1)

            def pair(i2, carry):
                cc = i2 * 2
                waitG(0)
                cp0 = startS(cc, 0)
                waitG(1)
                cp1 = startS(cc + 1, 1)
                cp0.wait()
                startG(cc + 2, 0)
                cp1.wait()

                @pl.when(cc + 3 < NCHUNK)
                def _():
                    startG(cc + 3, 1)

                return carry

            lax.fori_loop(0, (NCHUNK - 1) // 2, pair, 0)
            waitG(0)
            startS(NCHUNK - 1, 0).wait()
            plsc.subcore_barrier()

            @pl.when(c == 0)
            def _(g=g):
                pltpu.sync_copy(nb_sp.at[rs], outs[0][g].at[rs])

            @pl.when(c == 1)
            def _(g=g):
                pltpu.sync_copy(nb_sp.at[rs], outs[1][g].at[rs])

    return k(src_e, dst_e, g0, g1, z128)


def kernel(x, edge_index, batch, W_gat, b_gat, a_src, a_dst, W1, b1, W2, b2,
           Wf, bf):
    src = edge_index[0]
    dst = edge_index[1]
    eye = jnp.repeat(jnp.eye(HEADS, dtype=_f32), HID, axis=0)
    As = a_src.reshape(-1)[:, None] * eye
    Ad = a_dst.reshape(-1)[:, None] * eye

    h0, h1, ts, td = _tc1(x, W_gat, As, Ad)
    asf = ts.T.reshape(HEADS * N)
    adf = td.T.reshape(HEADS * N)

    znp = jnp.zeros((NPAD,), _f32)
    z128 = jnp.zeros((RPT, 128), _f32)
    ex, den = _sc_a(src, dst, asf, adf, znp)
    alpha = _sc_c(dst, ex, den)
    a00, a01, a10, a11 = _sc_d(src, dst, h0, h1, alpha, z128)
    g0, g1 = _tc2(a00, a10, a01, a11, b_gat.reshape(1, HEADS * HID))
    n00, n01, n10, n11 = _sc_e(src, dst, g0, g1, z128)
    emb = _tc3(g0, g1, n00, n01, n10, n11, batch.reshape(N, 1),
               W1[:128], W1[128:], b1.reshape(1, HID), W2, b2.reshape(1, HID),
               Wf, bf.reshape(1, OUT))
    return emb
